# baseline (speedup 1.0000x reference)
import functools
import math

import jax
import jax.numpy as jnp
from jax import lax
from jax.experimental import pallas as pl
from jax.experimental.pallas import tpu as pltpu
from jax.experimental.pallas import tpu_sc as plsc

N = 10000
E = 160000
B = 64
H = 4
C = 256

_NC = 2
_NS = 16
_NW = _NC * _NS

_R = 1280
_NROUND = 8
_NPAD = _R * _NROUND
_MW = H * C + 128


def _mm_body(a_ref, w_ref, b_ref, o_ref, *, act):
    x = jnp.dot(a_ref[...], w_ref[...], preferred_element_type=jnp.float32)
    x = x + b_ref[...]
    if act == "leaky":
        x = jnp.where(x >= 0, x, 0.01 * x)
    o_ref[...] = x


def _mm_ln_body(a_ref, w_ref, b_ref, g_ref, b2_ref, o_ref):
    x = jnp.dot(a_ref[...], w_ref[...], preferred_element_type=jnp.float32)
    x = x + b_ref[...]
    mu = jnp.mean(x, axis=-1, keepdims=True)
    var = jnp.mean((x - mu) ** 2, axis=-1, keepdims=True)
    o_ref[...] = (x - mu) * jax.lax.rsqrt(var + 1e-5) * g_ref[...] + b2_ref[...]


def _mm(a, w, b=None, act=None, ln=None, bm=1024):
    m, k = a.shape
    k2, n = w.shape
    assert k == k2, (a.shape, w.shape)
    if b is None:
        b = jnp.zeros((n,), jnp.float32)
    bm = min(bm, m)
    grid = (pl.cdiv(m, bm),)
    vec_spec = pl.BlockSpec((1, n), lambda i: (0, 0))
    in_specs = [
        pl.BlockSpec((bm, k), lambda i: (i, 0)),
        pl.BlockSpec((k, n), lambda i: (0, 0)),
        vec_spec,
    ]
    args = [a, w, b.reshape(1, n)]
    if ln is None:
        body = functools.partial(_mm_body, act=act)
    else:
        g, b2 = ln
        in_specs += [vec_spec, vec_spec]
        args += [g.reshape(1, n), b2.reshape(1, n)]
        body = _mm_ln_body
    return pl.pallas_call(
        body,
        grid=grid,
        in_specs=in_specs,
        out_specs=pl.BlockSpec((bm, n), lambda i: (i, 0)),
        out_shape=jax.ShapeDtypeStruct((m, n), jnp.float32),
        compiler_params=pltpu.CompilerParams(
            dimension_semantics=("parallel",)),
    )(*args)


def _emb_body(x_ref, emb_ref, g_ref, b_ref, o_ref):
    xi = x_ref[...]
    bm = xi.shape[0]
    oh = (xi == jax.lax.broadcasted_iota(jnp.int32, (bm, 128), 1))
    h = jnp.dot(oh.astype(jnp.float32), emb_ref[...],
                preferred_element_type=jnp.float32)
    mu = jnp.mean(h, axis=-1, keepdims=True)
    var = jnp.mean((h - mu) ** 2, axis=-1, keepdims=True)
    o_ref[...] = (h - mu) * jax.lax.rsqrt(var + 1e-5) * g_ref[...] + b_ref[...]


def _embed_ln(x, emb, g, b, bm=2000):
    emb_p = jnp.pad(emb, ((0, 128 - emb.shape[0]), (0, 0)))
    n, d = N, emb.shape[1]
    vec_spec = pl.BlockSpec((1, d), lambda i: (0, 0))
    return pl.pallas_call(
        _emb_body,
        grid=(pl.cdiv(n, bm),),
        in_specs=[
            pl.BlockSpec((bm, 1), lambda i: (i, 0)),
            pl.BlockSpec((128, d), lambda i: (0, 0)),
            vec_spec,
            vec_spec,
        ],
        out_specs=pl.BlockSpec((bm, d), lambda i: (i, 0)),
        out_shape=jax.ShapeDtypeStruct((n, d), jnp.float32),
        compiler_params=pltpu.CompilerParams(
            dimension_semantics=("parallel",)),
    )(x.astype(jnp.int32), emb_p, g.reshape(1, d), b.reshape(1, d))


def _sc_gather(table, idx, block=40):
    m = idx.shape[0]
    d = table.shape[1]
    per_w = m // _NW
    nblk = per_w // block
    assert per_w * _NW == m and nblk * block == per_w
    mesh = plsc.VectorSubcoreMesh(core_axis_name="c", subcore_axis_name="s")

    @functools.partial(
        pl.kernel, mesh=mesh,
        out_type=jax.ShapeDtypeStruct((m, d), jnp.float32),
        scratch_types=[
            pltpu.VMEM((block,), jnp.int32),
            pltpu.VMEM((block, d), jnp.float32),
            pltpu.SemaphoreType.DMA,
        ],
    )
    def gk(table_hbm, idx_hbm, out_hbm, idx_v, rows_v, sem):
        wid = lax.axis_index("s") * _NC + lax.axis_index("c")
        base = wid * per_w

        def body(i, _):
            off = base + i * block
            pltpu.sync_copy(idx_hbm.at[pl.ds(off, block)], idx_v)
            pltpu.async_copy(table_hbm.at[idx_v], rows_v, sem).wait()
            pltpu.sync_copy(rows_v, out_hbm.at[pl.ds(off, block)])
            return 0

        lax.fori_loop(0, nblk, body, 0)

    return gk(table, idx.astype(jnp.int32))


def _sc_msg_segsum(v, ee_s, w, src_s, dst_s, rb, zeros_hbm):
    mesh = plsc.VectorSubcoreMesh(core_axis_name="c", subcore_axis_name="s")

    @functools.partial(
        pl.kernel, mesh=mesh,
        out_type=jax.ShapeDtypeStruct((_NPAD, _MW), jnp.float32),
        scratch_types=[
            pltpu.VMEM((16, _MW), jnp.float32),
            pltpu.VMEM((16,), jnp.int32),
            pltpu.VMEM((16,), jnp.int32),
            pltpu.VMEM((16,), jnp.int32),
            pltpu.VMEM((16, H * C), jnp.float32),
            pltpu.VMEM((16, H * C), jnp.float32),
            pltpu.VMEM((16, 128), jnp.float32),
            pltpu.VMEM((16, _MW), jnp.float32),
            pltpu.VMEM((16,), jnp.int32),
        ],
    )
    def sk(v_hbm, ee_hbm, w_hbm, src_hbm, dst_hbm, rb_hbm, z_hbm, out_hbm,
           zbuf, dstv, srcv, idxv, vbuf, ebuf, wbuf, mbuf, rbs):
        cid = lax.axis_index("c")
        sid = lax.axis_index("s")
        wid = sid * _NC + cid
        pltpu.sync_copy(rb_hbm.at[pl.ds(cid * 16, 16)], rbs)
        rbv = rbs[...]
        pltpu.sync_copy(z_hbm, zbuf)

        for j in range(_NROUND // _NC):
            r = cid * (_NROUND // _NC) + j
            lo = rbv[j]
            hi = rbv[j + 1]
            base = r * _R
            for z in range(_R // 16 // 16):
                pltpu.sync_copy(
                    zbuf,
                    out_hbm.at[pl.ds(base + sid * (_R // 16) + z * 16, 16)])
            plsc.subcore_barrier()
            lo_al = (lo // 16) * 16
            nblk = (hi - lo_al + 15) // 16
            ntile = (nblk - sid + 15) // 16

            def blk(t, _):
                e0 = lo_al + (sid + t * 16) * 16
                pltpu.sync_copy(dst_hbm.at[pl.ds(e0, 16)], dstv)
                pltpu.sync_copy(src_hbm.at[pl.ds(e0, 16)], srcv)
                pltpu.sync_copy(v_hbm.at[srcv], vbuf)
                pltpu.sync_copy(ee_hbm.at[pl.ds(e0, 16)], ebuf)
                pltpu.sync_copy(w_hbm.at[pl.ds(e0, 16)], wbuf)
                def edge(e, _):
                    wrow = wbuf[e, pl.ds(0, 16)]
                    mbuf[e, pl.ds(H * C, 16)] = wrow
                    for h in range(H):
                        weh = wrow[h]
                        for s in range(C // 16):
                            cs = pl.ds(h * C + s * 16, 16)
                            mbuf[e, cs] = (vbuf[e, cs] + ebuf[e, cs]) * weh
                    return 0

                lax.fori_loop(0, 16, edge, 0)
                dvec = dstv[...]
                inr = (dvec >= base) & (dvec < base + _R)
                idxv[...] = jnp.where(inr, dvec, N + wid)
                pltpu.sync_copy(mbuf, out_hbm.at[idxv], add=True)
                return 0

            lax.fori_loop(0, ntile, blk, 0)

    return sk(v, ee_s, w, src_s, dst_s, rb, zeros_hbm)


def _alpha_w_body(qd_ref, ks_ref, ee_ref, o_ref):
    qd = qd_ref[...]
    ks = ks_ref[...]
    ee = ee_ref[...]
    bm = qd.shape[0]
    ws = []
    for h in range(H):
        s = slice(h * C, (h + 1) * C)
        kj = ks[:, s] + ee[:, s]
        alpha = jnp.sum(qd[:, s] * kj, axis=-1, keepdims=True) / math.sqrt(C)
        ws.append(jnp.exp(alpha))
    o_ref[...] = jnp.concatenate(
        ws + [jnp.zeros((bm, 128 - H), jnp.float32)], axis=1)


def _alpha_w(qd, ks, ee, bm=512):
    spec = pl.BlockSpec((bm, H * C), lambda i: (i, 0))
    return pl.pallas_call(
        _alpha_w_body,
        grid=(pl.cdiv(E, bm),),
        in_specs=[spec, spec, spec],
        out_specs=pl.BlockSpec((bm, 128), lambda i: (i, 0)),
        out_shape=jax.ShapeDtypeStruct((E, 128), jnp.float32),
        compiler_params=pltpu.CompilerParams(
            dimension_semantics=("parallel",)),
    )(qd, ks, ee)


def _final_body(h_ref, ws_ref, oe_ref, o_ref):
    x = jnp.dot(h_ref[...], ws_ref[...], preferred_element_type=jnp.float32)
    oe = oe_ref[...]
    for h in range(H):
        num = oe[:, h * C:(h + 1) * C]
        den = oe[:, H * C + h][:, None] + 1e-16
        x = x + (1.0 / H) * num / den
    o_ref[...] = jnp.where(x >= 0, x, 0.01 * x)


def _layer_final(h, Ws, out_ext, bm=1000):
    return pl.pallas_call(
        _final_body,
        grid=(pl.cdiv(N, bm),),
        in_specs=[
            pl.BlockSpec((bm, C), lambda i: (i, 0)),
            pl.BlockSpec((C, C), lambda i: (0, 0)),
            pl.BlockSpec((bm, _MW), lambda i: (i, 0)),
        ],
        out_specs=pl.BlockSpec((bm, C), lambda i: (i, 0)),
        out_shape=jax.ShapeDtypeStruct((N, C), jnp.float32),
        compiler_params=pltpu.CompilerParams(
            dimension_semantics=("parallel",)),
    )(h, Ws, out_ext)


def _tconv_layer(h, src_s, dst_s, e_s, rb, zeros_hbm, Wq, Wk, Wv, We, Ws):
    q = _mm(h, Wq)
    k = _mm(h, Wk)
    v = _mm(h, Wv)
    ee = _mm(e_s, We)
    qd = _sc_gather(q, dst_s)
    ks = _sc_gather(k, src_s)
    w = _alpha_w(qd, ks, ee)
    out_ext = _sc_msg_segsum(v, ee, w, src_s, dst_s, rb, zeros_hbm)[:N]
    return _layer_final(h, Ws, out_ext)


def kernel(x, edge_index, edge_attr, energies, batch, emb, node_ln_g,
           node_ln_b, eW1, eb1, eW2, eb2, edge_ln_g, edge_ln_b, enW1, enb1,
           enW2, enb2, fcW1, fcb1, fcW2, fcb2, tc_Wq, tc_Wk, tc_Wv, tc_We,
           tc_Ws):
    h = _embed_ln(x, emb, node_ln_g, node_ln_b)

    ea_p = jnp.pad(edge_attr, ((0, 0), (0, 128 - edge_attr.shape[1])))
    eW1_p = jnp.pad(eW1, ((0, 128 - eW1.shape[0]), (0, 0)))
    e1 = _mm(ea_p, eW1_p, eb1, act="leaky")
    e = _mm(e1, eW2, eb2, ln=(edge_ln_g, edge_ln_b))

    src = edge_index[0].astype(jnp.int32)
    dst = edge_index[1].astype(jnp.int32)
    perm = jnp.argsort(dst)
    dst_s = dst[perm]
    src_s = src[perm]
    e_s = _sc_gather(e, perm.astype(jnp.int32))
    rb9 = jnp.searchsorted(dst_s, jnp.arange(_NROUND + 1) * _R).astype(
        jnp.int32)
    nr_c = _NROUND // _NC
    rb = jnp.concatenate(
        [jnp.pad(rb9[c * nr_c:c * nr_c + nr_c + 1], (0, 16 - (nr_c + 1)))
         for c in range(_NC)])
    zeros_hbm = jnp.zeros((16, _MW), jnp.float32)

    for l in range(3):
        h = _tconv_layer(h, src_s, dst_s, e_s, rb, zeros_hbm, tc_Wq[l],
                         tc_Wk[l], tc_Wv[l], tc_We[l], tc_Ws[l])

    counts = jax.ops.segment_sum(jnp.ones((N,), jnp.float32), batch,
                                 num_segments=B)
    ssum = jax.ops.segment_sum(h, batch, num_segments=B)
    smean = ssum / jnp.clip(counts, 1.0)[:, None]
    smax = jax.ops.segment_max(h, batch, num_segments=B)
    smax = jnp.where(jnp.isfinite(smax), smax, 0.0)

    en1 = _mm(jnp.pad(energies, ((0, 0), (0, 55))),
              jnp.pad(enW1, ((0, 55), (0, 0))), enb1, act="leaky")
    en = _mm(en1, enW2, enb2)

    feat = jnp.concatenate([ssum, smean, smax, en], axis=-1)
    f1 = _mm(feat, fcW1, fcb1, act="leaky")
    out = _mm(f1, jnp.pad(fcW2, ((0, 0), (0, 92))),
              jnp.pad(fcb2, (0, 92)))[:, :804]
    return out


# speedup vs baseline: 1.5174x; 1.5174x over previous
import functools
import math

import jax
import jax.numpy as jnp
from jax import lax
from jax.experimental import pallas as pl
from jax.experimental.pallas import tpu as pltpu
from jax.experimental.pallas import tpu_sc as plsc

N = 10000
E = 160000
B = 64
H = 4
C = 256

_NC = 2
_NS = 16
_NW = _NC * _NS

_R = 1280
_NROUND = 8
_NPAD = _R * _NROUND
_MW = H * C + 128


def _mm_body(a_ref, w_ref, b_ref, o_ref, *, act):
    x = jnp.dot(a_ref[...], w_ref[...], preferred_element_type=jnp.float32)
    x = x + b_ref[...]
    if act == "leaky":
        x = jnp.where(x >= 0, x, 0.01 * x)
    o_ref[...] = x


def _mm_ln_body(a_ref, w_ref, b_ref, g_ref, b2_ref, o_ref):
    x = jnp.dot(a_ref[...], w_ref[...], preferred_element_type=jnp.float32)
    x = x + b_ref[...]
    mu = jnp.mean(x, axis=-1, keepdims=True)
    var = jnp.mean((x - mu) ** 2, axis=-1, keepdims=True)
    o_ref[...] = (x - mu) * jax.lax.rsqrt(var + 1e-5) * g_ref[...] + b2_ref[...]


def _mm(a, w, b=None, act=None, ln=None, bm=1024):
    m, k = a.shape
    k2, n = w.shape
    assert k == k2, (a.shape, w.shape)
    if b is None:
        b = jnp.zeros((n,), jnp.float32)
    bm = min(bm, m)
    grid = (pl.cdiv(m, bm),)
    vec_spec = pl.BlockSpec((1, n), lambda i: (0, 0))
    in_specs = [
        pl.BlockSpec((bm, k), lambda i: (i, 0)),
        pl.BlockSpec((k, n), lambda i: (0, 0)),
        vec_spec,
    ]
    args = [a, w, b.reshape(1, n)]
    if ln is None:
        body = functools.partial(_mm_body, act=act)
    else:
        g, b2 = ln
        in_specs += [vec_spec, vec_spec]
        args += [g.reshape(1, n), b2.reshape(1, n)]
        body = _mm_ln_body
    return pl.pallas_call(
        body,
        grid=grid,
        in_specs=in_specs,
        out_specs=pl.BlockSpec((bm, n), lambda i: (i, 0)),
        out_shape=jax.ShapeDtypeStruct((m, n), jnp.float32),
        compiler_params=pltpu.CompilerParams(
            dimension_semantics=("parallel",)),
    )(*args)


def _emb_body(x_ref, emb_ref, g_ref, b_ref, o_ref):
    xi = x_ref[...]
    bm = xi.shape[0]
    oh = (xi == jax.lax.broadcasted_iota(jnp.int32, (bm, 128), 1))
    h = jnp.dot(oh.astype(jnp.float32), emb_ref[...],
                preferred_element_type=jnp.float32)
    mu = jnp.mean(h, axis=-1, keepdims=True)
    var = jnp.mean((h - mu) ** 2, axis=-1, keepdims=True)
    o_ref[...] = (h - mu) * jax.lax.rsqrt(var + 1e-5) * g_ref[...] + b_ref[...]


def _embed_ln(x, emb, g, b, bm=2000):
    emb_p = jnp.pad(emb, ((0, 128 - emb.shape[0]), (0, 0)))
    n, d = N, emb.shape[1]
    vec_spec = pl.BlockSpec((1, d), lambda i: (0, 0))
    return pl.pallas_call(
        _emb_body,
        grid=(pl.cdiv(n, bm),),
        in_specs=[
            pl.BlockSpec((bm, 1), lambda i: (i, 0)),
            pl.BlockSpec((128, d), lambda i: (0, 0)),
            vec_spec,
            vec_spec,
        ],
        out_specs=pl.BlockSpec((bm, d), lambda i: (i, 0)),
        out_shape=jax.ShapeDtypeStruct((n, d), jnp.float32),
        compiler_params=pltpu.CompilerParams(
            dimension_semantics=("parallel",)),
    )(x.astype(jnp.int32), emb_p, g.reshape(1, d), b.reshape(1, d))


def _sc_gather(table, idx, block=40):
    m = idx.shape[0]
    d = table.shape[1]
    per_w = m // _NW
    nblk = per_w // block
    npair = nblk // 2
    tail = nblk - 2 * npair
    assert per_w * _NW == m and nblk * block == per_w and block % 8 == 0
    mesh = plsc.VectorSubcoreMesh(core_axis_name="c", subcore_axis_name="s")

    @functools.partial(
        pl.kernel, mesh=mesh,
        out_type=jax.ShapeDtypeStruct((m, d), jnp.float32),
        scratch_types=[
            pltpu.VMEM((block,), jnp.int32),
            pltpu.VMEM((block,), jnp.int32),
            pltpu.VMEM((block, d), jnp.float32),
            pltpu.VMEM((block, d), jnp.float32),
            pltpu.SemaphoreType.DMA,
            pltpu.SemaphoreType.DMA,
            pltpu.SemaphoreType.DMA,
            pltpu.SemaphoreType.DMA,
        ],
    )
    def gk(table_hbm, idx_hbm, out_hbm, i0, i1, r0, r1, sg0, sg1, sw0, sw1):
        wid = lax.axis_index("s") * _NC + lax.axis_index("c")
        base = wid * per_w
        last = base + (npair - 1) * 2 * block

        pltpu.sync_copy(idx_hbm.at[pl.ds(base, block)], i0)
        pltpu.async_copy(table_hbm.at[i0], r0, sg0)
        pltpu.sync_copy(idx_hbm.at[pl.ds(base + block, block)], i1)
        pltpu.async_copy(table_hbm.at[i1], r1, sg1)

        def body(t, _):
            off_a = base + (2 * t) * block
            off_b = off_a + block
            nxt = jnp.minimum(off_a + 2 * block, last)
            pltpu.make_async_copy(table_hbm.at[i0], r0, sg0).wait()
            pltpu.async_copy(r0, out_hbm.at[pl.ds(off_a, block)], sw0)
            pltpu.make_async_copy(table_hbm.at[i1], r1, sg1).wait()
            pltpu.async_copy(r1, out_hbm.at[pl.ds(off_b, block)], sw1)
            pltpu.sync_copy(idx_hbm.at[pl.ds(nxt, block)], i0)
            pltpu.make_async_copy(
                r0, out_hbm.at[pl.ds(off_a, block)], sw0).wait()
            pltpu.async_copy(table_hbm.at[i0], r0, sg0)
            pltpu.sync_copy(idx_hbm.at[pl.ds(nxt + block, block)], i1)
            pltpu.make_async_copy(
                r1, out_hbm.at[pl.ds(off_b, block)], sw1).wait()
            pltpu.async_copy(table_hbm.at[i1], r1, sg1)
            return 0

        lax.fori_loop(0, npair, body, 0)
        pltpu.make_async_copy(table_hbm.at[i0], r0, sg0).wait()
        pltpu.make_async_copy(table_hbm.at[i1], r1, sg1).wait()
        if tail:
            off = base + 2 * npair * block
            pltpu.sync_copy(idx_hbm.at[pl.ds(off, block)], i0)
            pltpu.async_copy(table_hbm.at[i0], r0, sg0).wait()
            pltpu.sync_copy(r0, out_hbm.at[pl.ds(off, block)])

    return gk(table, idx.astype(jnp.int32))


def _sc_segsum(m_ext, dst_s, rb, zeros_hbm):
    mesh = plsc.VectorSubcoreMesh(core_axis_name="c", subcore_axis_name="s")

    @functools.partial(
        pl.kernel, mesh=mesh,
        out_type=jax.ShapeDtypeStruct((_NPAD, _MW), jnp.float32),
        scratch_types=[
            pltpu.VMEM((_R // 16, _MW), jnp.float32),
            pltpu.VMEM((16,), jnp.int32),
            pltpu.VMEM((16,), jnp.int32),
            pltpu.VMEM((16, _MW), jnp.float32),
            pltpu.VMEM((16,), jnp.int32),
        ],
    )
    def sk(m_hbm, dst_hbm, rb_hbm, z_hbm, out_hbm, zbuf, dstv, idxv, mbuf,
           rbs):
        cid = lax.axis_index("c")
        sid = lax.axis_index("s")
        wid = sid * _NC + cid
        pltpu.sync_copy(rb_hbm.at[pl.ds(cid * 16, 16)], rbs)
        rbv = rbs[...]
        pltpu.sync_copy(z_hbm, zbuf)

        for j in range(_NROUND // _NC):
            r = cid * (_NROUND // _NC) + j
            lo = rbv[j]
            hi = rbv[j + 1]
            base = r * _R
            pltpu.sync_copy(zbuf,
                            out_hbm.at[pl.ds(base + sid * (_R // 16),
                                             _R // 16)])
            plsc.subcore_barrier()
            lo_al = (lo // 16) * 16
            nblk = (hi - lo_al + 15) // 16
            ntile = (nblk - sid + 15) // 16

            def blk(t, _):
                e0 = lo_al + (sid + t * 16) * 16
                pltpu.sync_copy(dst_hbm.at[pl.ds(e0, 16)], dstv)
                dvec = dstv[...]
                inr = (dvec >= base) & (dvec < base + _R)
                idxv[...] = jnp.where(inr, dvec, N + wid)
                pltpu.sync_copy(m_hbm.at[pl.ds(e0, 16)], mbuf)
                pltpu.sync_copy(mbuf, out_hbm.at[idxv], add=True)
                return 0

            lax.fori_loop(0, ntile, blk, 0)

    return sk(m_ext, dst_s, rb, zeros_hbm)


def _alpha_m_body(qd_ref, ks_ref, vs_ref, ee_ref, o_ref):
    qd = qd_ref[...]
    ks = ks_ref[...]
    vs = vs_ref[...]
    ee = ee_ref[...]
    bm = qd.shape[0]
    ws = []
    for h in range(H):
        s = slice(h * C, (h + 1) * C)
        kj = ks[:, s] + ee[:, s]
        alpha = jnp.sum(qd[:, s] * kj, axis=-1, keepdims=True) / math.sqrt(C)
        w = jnp.exp(alpha)
        o_ref[:, s] = (vs[:, s] + ee[:, s]) * w
        ws.append(w)
    tail = jnp.concatenate(ws + [jnp.zeros((bm, 128 - H), jnp.float32)],
                           axis=1)
    o_ref[:, H * C:] = tail


def _alpha_m(qd, ks, vs, ee, bm=512):
    spec = pl.BlockSpec((bm, H * C), lambda i: (i, 0))
    return pl.pallas_call(
        _alpha_m_body,
        grid=(pl.cdiv(E, bm),),
        in_specs=[spec, spec, spec, spec],
        out_specs=pl.BlockSpec((bm, _MW), lambda i: (i, 0)),
        out_shape=jax.ShapeDtypeStruct((E, _MW), jnp.float32),
        compiler_params=pltpu.CompilerParams(
            dimension_semantics=("parallel",)),
    )(qd, ks, vs, ee)


def _final_body(h_ref, ws_ref, oe_ref, o_ref):
    x = jnp.dot(h_ref[...], ws_ref[...], preferred_element_type=jnp.float32)
    oe = oe_ref[...]
    for h in range(H):
        num = oe[:, h * C:(h + 1) * C]
        den = oe[:, H * C + h][:, None] + 1e-16
        x = x + (1.0 / H) * num / den
    o_ref[...] = jnp.where(x >= 0, x, 0.01 * x)


def _layer_final(h, Ws, out_ext, bm=1000):
    return pl.pallas_call(
        _final_body,
        grid=(pl.cdiv(N, bm),),
        in_specs=[
            pl.BlockSpec((bm, C), lambda i: (i, 0)),
            pl.BlockSpec((C, C), lambda i: (0, 0)),
            pl.BlockSpec((bm, _MW), lambda i: (i, 0)),
        ],
        out_specs=pl.BlockSpec((bm, C), lambda i: (i, 0)),
        out_shape=jax.ShapeDtypeStruct((N, C), jnp.float32),
        compiler_params=pltpu.CompilerParams(
            dimension_semantics=("parallel",)),
    )(h, Ws, out_ext)


def _tconv_layer(h, src_s, dst_s, e_s, rb, zeros_hbm, Wq, Wk, Wv, We, Ws):
    q = _mm(h, Wq)
    k = _mm(h, Wk)
    v = _mm(h, Wv)
    ee = _mm(e_s, We)
    qd = _sc_gather(q, dst_s)
    ks = _sc_gather(k, src_s)
    vs = _sc_gather(v, src_s)
    m_ext = _alpha_m(qd, ks, vs, ee)
    out_ext = _sc_segsum(m_ext, dst_s, rb, zeros_hbm)[:N]
    return _layer_final(h, Ws, out_ext)


def kernel(x, edge_index, edge_attr, energies, batch, emb, node_ln_g,
           node_ln_b, eW1, eb1, eW2, eb2, edge_ln_g, edge_ln_b, enW1, enb1,
           enW2, enb2, fcW1, fcb1, fcW2, fcb2, tc_Wq, tc_Wk, tc_Wv, tc_We,
           tc_Ws):
    h = _embed_ln(x, emb, node_ln_g, node_ln_b)

    ea_p = jnp.pad(edge_attr, ((0, 0), (0, 128 - edge_attr.shape[1])))
    eW1_p = jnp.pad(eW1, ((0, 128 - eW1.shape[0]), (0, 0)))
    e1 = _mm(ea_p, eW1_p, eb1, act="leaky")
    e = _mm(e1, eW2, eb2, ln=(edge_ln_g, edge_ln_b))

    src = edge_index[0].astype(jnp.int32)
    dst = edge_index[1].astype(jnp.int32)
    perm = jnp.argsort(dst)
    dst_s = dst[perm]
    src_s = src[perm]
    e_s = _sc_gather(e, perm.astype(jnp.int32))
    rb9 = jnp.searchsorted(dst_s, jnp.arange(_NROUND + 1) * _R).astype(
        jnp.int32)
    nr_c = _NROUND // _NC
    rb = jnp.concatenate(
        [jnp.pad(rb9[c * nr_c:c * nr_c + nr_c + 1], (0, 16 - (nr_c + 1)))
         for c in range(_NC)])
    zeros_hbm = jnp.zeros((_R // 16, _MW), jnp.float32)

    for l in range(3):
        h = _tconv_layer(h, src_s, dst_s, e_s, rb, zeros_hbm, tc_Wq[l],
                         tc_Wk[l], tc_Wv[l], tc_We[l], tc_Ws[l])

    counts = jax.ops.segment_sum(jnp.ones((N,), jnp.float32), batch,
                                 num_segments=B)
    ssum = jax.ops.segment_sum(h, batch, num_segments=B)
    smean = ssum / jnp.clip(counts, 1.0)[:, None]
    smax = jax.ops.segment_max(h, batch, num_segments=B)
    smax = jnp.where(jnp.isfinite(smax), smax, 0.0)

    en1 = _mm(jnp.pad(energies, ((0, 0), (0, 55))),
              jnp.pad(enW1, ((0, 55), (0, 0))), enb1, act="leaky")
    en = _mm(en1, enW2, enb2)

    feat = jnp.concatenate([ssum, smean, smax, en], axis=-1)
    f1 = _mm(feat, fcW1, fcb1, act="leaky")
    out = _mm(f1, jnp.pad(fcW2, ((0, 0), (0, 92))),
              jnp.pad(fcb2, (0, 92)))[:, :804]
    return out
